# Initial kernel scaffold; baseline (speedup 1.0000x reference)
import jax
import jax.numpy as jnp
from jax.experimental import pallas as pl


def kernel(entity_emb, user_emb, item_emb_cf, edge_index, edge_type, mat_row, mat_col, mat_val, relation_weight, gate1_w, gate2_w):
    raise NotImplementedError("write your pallas kernel here")


# speedup vs baseline: 2.3002x; 2.3002x over previous
import functools

import jax
import jax.numpy as jnp
from jax import lax
from jax.experimental import pallas as pl
from jax.experimental.pallas import tpu as pltpu
from jax.experimental.pallas import tpu_sc as plsc

N_ENTITIES = 50000
N_USERS = 30000
N_ITEMS = 20000
N_EDGES = 800000
NNZ = 600000
DIM = 64
MIN_NORM = 1e-15
EPS = 1e-5

NC = 2
NS = 16
NW = NC * NS
CHUNK = 128


def _pad_to(n, m):
    return ((n + m - 1) // m) * m


E_PAD = _pad_to(N_EDGES, NW * CHUNK)
Z_PAD = _pad_to(NNZ, NW * CHUNK)
ENT_PAD = _pad_to(N_ENTITIES + 1, NS)
ITM_PAD = _pad_to(N_ITEMS + 1, NS)
USR_PAD = _pad_to(N_USERS + 1, NS)


def _sc_gather(table, idx):
    n = idx.shape[0]
    d = table.shape[1]
    per_w = n // NW
    nch = per_w // CHUNK
    mesh = plsc.VectorSubcoreMesh(core_axis_name="c", subcore_axis_name="s")

    @functools.partial(
        pl.kernel, mesh=mesh,
        out_type=jax.ShapeDtypeStruct((n, d), jnp.float32),
        compiler_params=pltpu.CompilerParams(use_tc_tiling_on_sc=False),
        scratch_types=[
            pltpu.VMEM((CHUNK,), jnp.int32),
            pltpu.VMEM((CHUNK, d), jnp.float32),
            pltpu.SemaphoreType.DMA,
        ],
    )
    def k(table_h, idx_h, out_h, idx_v, rows_v, sem):
        wid = lax.axis_index("s") * NC + lax.axis_index("c")
        base = wid * per_w

        def step(i, carry):
            off = base + i * CHUNK
            pltpu.sync_copy(idx_h.at[pl.ds(off, CHUNK)], idx_v)
            pltpu.async_copy(table_h.at[idx_v], rows_v, sem).wait()
            pltpu.sync_copy(rows_v, out_h.at[pl.ds(off, CHUNK)])
            return carry

        lax.fori_loop(0, nch, step, 0)

    return k(table, idx)


def _sc_scatter_add(src, idx, nseg_pad, zeros_h):
    n, d = src.shape
    per_w = n // NW
    nch = per_w // CHUNK
    rz = nseg_pad // NS
    mesh = plsc.VectorSubcoreMesh(core_axis_name="c", subcore_axis_name="s")

    @functools.partial(
        pl.kernel, mesh=mesh,
        out_type=jax.ShapeDtypeStruct((NC, nseg_pad, d), jnp.float32),
        compiler_params=pltpu.CompilerParams(use_tc_tiling_on_sc=False),
        scratch_types=[
            pltpu.VMEM((CHUNK,), jnp.int32),
            pltpu.VMEM((CHUNK, d), jnp.float32),
            pltpu.VMEM_SHARED((nseg_pad, d), jnp.float32),
        ],
    )
    def k(src_h, idx_h, z_h, out_h, idx_v, rows_v, acc):
        cid = lax.axis_index("c")
        sid = lax.axis_index("s")
        wid = sid * NC + cid
        base = wid * per_w
        pltpu.sync_copy(z_h.at[pl.ds(sid * rz, rz)], acc.at[pl.ds(sid * rz, rz)])
        plsc.subcore_barrier()

        def step(i, carry):
            off = base + i * CHUNK
            pltpu.sync_copy(idx_h.at[pl.ds(off, CHUNK)], idx_v)
            pltpu.sync_copy(src_h.at[pl.ds(off, CHUNK)], rows_v)
            pltpu.sync_copy(rows_v, acc.at[idx_v], add=True)
            return carry

        lax.fori_loop(0, nch, step, 0)
        plsc.subcore_barrier()
        pltpu.sync_copy(acc.at[pl.ds(sid * rz, rz)],
                        out_h.at[cid, pl.ds(sid * rz, rz)])

    return k(src, idx, zeros_h)


def _sc_count(idx, n, nseg_pad, zeros_h, ones_h):
    per_w = n // NW
    nch = per_w // CHUNK
    rz = nseg_pad // NS
    mesh = plsc.VectorSubcoreMesh(core_axis_name="c", subcore_axis_name="s")

    @functools.partial(
        pl.kernel, mesh=mesh,
        out_type=jax.ShapeDtypeStruct((NC, nseg_pad, 16), jnp.float32),
        compiler_params=pltpu.CompilerParams(use_tc_tiling_on_sc=False),
        scratch_types=[
            pltpu.VMEM((CHUNK,), jnp.int32),
            pltpu.VMEM((CHUNK, 16), jnp.float32),
            pltpu.VMEM_SHARED((nseg_pad, 16), jnp.float32),
        ],
    )
    def k(idx_h, z_h, ones_hh, out_h, idx_v, ones_v, acc):
        cid = lax.axis_index("c")
        sid = lax.axis_index("s")
        wid = sid * NC + cid
        base = wid * per_w
        pltpu.sync_copy(ones_hh, ones_v)
        pltpu.sync_copy(z_h.at[pl.ds(sid * rz, rz)], acc.at[pl.ds(sid * rz, rz)])
        plsc.subcore_barrier()

        def step(i, carry):
            off = base + i * CHUNK
            pltpu.sync_copy(idx_h.at[pl.ds(off, CHUNK)], idx_v)
            pltpu.sync_copy(ones_v, acc.at[idx_v], add=True)
            return carry

        lax.fori_loop(0, nch, step, 0)
        plsc.subcore_barrier()
        pltpu.sync_copy(acc.at[pl.ds(sid * rz, rz)],
                        out_h.at[cid, pl.ds(sid * rz, rz)])

    return k(idx, zeros_h, ones_h)


def _norm(x):
    return jnp.clip(jnp.sqrt(jnp.sum(x * x, axis=-1, keepdims=True)), MIN_NORM, None)


def _artanh(x):
    x = jnp.clip(x, -1.0 + 1e-7, 1.0 - 1e-7)
    return 0.5 * jnp.log((1.0 + x) / (1.0 - x))


def _project(x):
    norm = _norm(x)
    maxnorm = 1.0 - EPS
    return jnp.where(norm > maxnorm, x / norm * maxnorm, x)


def _expmap0(u):
    norm = _norm(u)
    return _project(jnp.tanh(norm) * u / norm)


def _mobius_add(x, y):
    x2 = jnp.sum(x * x, axis=-1, keepdims=True)
    y2 = jnp.sum(y * y, axis=-1, keepdims=True)
    xy = jnp.sum(x * y, axis=-1, keepdims=True)
    num = (1.0 + 2.0 * xy + y2) * x + (1.0 - x2) * y
    den = 1.0 + 2.0 * xy + x2 * y2
    return num / jnp.clip(den, MIN_NORM, None)


def _lambda_x(x):
    x2 = jnp.sum(x * x, axis=-1, keepdims=True)
    return 2.0 / jnp.clip(1.0 - x2, MIN_NORM, None)


def _expmap(u, p):
    nu = _norm(u)
    second = jnp.tanh(_lambda_x(p) * nu / 2.0) * u / nu
    return _project(_mobius_add(p, second))


def _logmap(y, x):
    sub = _mobius_add(-x, y)
    ns = _norm(sub)
    return 2.0 / _lambda_x(x) * _artanh(ns) * sub / ns


def _edge_math(head_rows, tail_rows, rel_rows):
    blk = 1024
    grid = E_PAD // blk

    def body(h_ref, t_ref, r_ref, lo_ref, hi_ref):
        h = h_ref[...]
        t = t_ref[...]
        r = r_ref[...]
        hh = _expmap0(h)
        ht = _expmap(t, hh)
        hr = _expmap(r, hh)
        res = _project(_mobius_add(ht, hr))
        res = _logmap(res, hh)
        lo_ref[...] = res[:, :32]
        hi_ref[...] = res[:, 32:]

    spec_in = pl.BlockSpec((blk, DIM), lambda i: (i, 0))
    spec_out = pl.BlockSpec((blk, 32), lambda i: (i, 0))
    return pl.pallas_call(
        body, grid=(grid,),
        in_specs=[spec_in, spec_in, spec_in],
        out_specs=[spec_out, spec_out],
        out_shape=[jax.ShapeDtypeStruct((E_PAD, 32), jnp.float32)] * 2,
    )(head_rows, tail_rows, rel_rows)


def _gate_fusion(item_cf, item_kg, g1, g2):
    blk = 1000
    grid = N_ITEMS // blk

    def body(cf_ref, kg_ref, g1_ref, g2_ref, out_ref):
        cf = cf_ref[...]
        kg = kg_ref[...]
        y = lax.dot_general(cf, g1_ref[...], (((1,), (1,)), ((), ())),
                            preferred_element_type=jnp.float32)
        y += lax.dot_general(kg, g2_ref[...], (((1,), (1,)), ((), ())),
                             preferred_element_type=jnp.float32)
        gi = jax.nn.sigmoid(y)
        out_ref[...] = gi * cf + (1.0 - gi) * kg

    spec = pl.BlockSpec((blk, DIM), lambda i: (i, 0))
    wspec = pl.BlockSpec((DIM, DIM), lambda i: (0, 0))
    return pl.pallas_call(
        body, grid=(grid,),
        in_specs=[spec, spec, wspec, wspec],
        out_specs=spec,
        out_shape=jax.ShapeDtypeStruct((N_ITEMS, DIM), jnp.float32),
    )(item_cf, item_kg, g1, g2)


def _scale_rows(rows, vals):
    blk = 2048
    grid = Z_PAD // blk

    def body(r_ref, v_ref, o_ref):
        o_ref[...] = r_ref[...] * v_ref[...]

    return pl.pallas_call(
        body, grid=(grid,),
        in_specs=[pl.BlockSpec((blk, DIM), lambda i: (i, 0)),
                  pl.BlockSpec((blk, 1), lambda i: (i, 0))],
        out_specs=pl.BlockSpec((blk, DIM), lambda i: (i, 0)),
        out_shape=jax.ShapeDtypeStruct((Z_PAD, DIM), jnp.float32),
    )(rows, vals)


def _finalize_entity(lo_p, hi_p, cnt_p):
    blk = 1000
    grid = N_ENTITIES // blk

    def body(lo_ref, hi_ref, c_ref, o_ref):
        lo = lo_ref[0] + lo_ref[1]
        hi = hi_ref[0] + hi_ref[1]
        c = c_ref[0, :, 0:1] + c_ref[1, :, 0:1]
        s = jnp.concatenate([lo, hi], axis=1)
        o_ref[...] = s / jnp.clip(c, 1.0, None)

    return pl.pallas_call(
        body, grid=(grid,),
        in_specs=[pl.BlockSpec((NC, blk, 32), lambda i: (0, i, 0)),
                  pl.BlockSpec((NC, blk, 32), lambda i: (0, i, 0)),
                  pl.BlockSpec((NC, blk, 16), lambda i: (0, i, 0))],
        out_specs=pl.BlockSpec((blk, DIM), lambda i: (i, 0)),
        out_shape=jax.ShapeDtypeStruct((N_ENTITIES, DIM), jnp.float32),
    )(lo_p, hi_p, cnt_p)


def _sum_partials(p, nrows):
    blk = 1000
    grid = nrows // blk

    def body(p_ref, o_ref):
        o_ref[...] = p_ref[0] + p_ref[1]

    return pl.pallas_call(
        body, grid=(grid,),
        in_specs=[pl.BlockSpec((NC, blk, DIM), lambda i: (0, i, 0))],
        out_specs=pl.BlockSpec((blk, DIM), lambda i: (i, 0)),
        out_shape=jax.ShapeDtypeStruct((nrows, DIM), jnp.float32),
    )(p)


def kernel(entity_emb, user_emb, item_emb_cf, edge_index, edge_type,
           mat_row, mat_col, mat_val, relation_weight, gate1_w, gate2_w):
    i32 = jnp.int32
    head = edge_index[0].astype(i32)
    tail = edge_index[1].astype(i32)
    rel = (edge_type - 1).astype(i32)
    mrow = mat_row.astype(i32)
    mcol = mat_col.astype(i32)

    ep = E_PAD - N_EDGES
    zp = Z_PAD - NNZ
    head_g = jnp.concatenate([head, jnp.zeros((ep,), i32)])
    tail_g = jnp.concatenate([tail, jnp.zeros((ep,), i32)])
    rel_g = jnp.concatenate([rel, jnp.zeros((ep,), i32)])
    head_s = jnp.concatenate([head, jnp.full((ep,), N_ENTITIES, i32)])
    mrow_g = jnp.concatenate([mrow, jnp.zeros((zp,), i32)])
    mcol_g = jnp.concatenate([mcol, jnp.zeros((zp,), i32)])
    mrow_s = jnp.concatenate([mrow, jnp.full((zp,), N_USERS, i32)])
    mcol_s = jnp.concatenate([mcol, jnp.full((zp,), N_ITEMS, i32)])
    val_p = jnp.concatenate([mat_val, jnp.zeros((zp,), jnp.float32)])[:, None]

    z_ent32 = jnp.zeros((ENT_PAD, 32), jnp.float32)
    z_ent16 = jnp.zeros((ENT_PAD, 16), jnp.float32)
    z_itm = jnp.zeros((ITM_PAD, DIM), jnp.float32)
    z_usr = jnp.zeros((USR_PAD, DIM), jnp.float32)
    ones16 = jnp.ones((CHUNK, 16), jnp.float32)

    head_rows = _sc_gather(entity_emb, head_g)
    tail_rows = _sc_gather(entity_emb, tail_g)
    rel_rows = _sc_gather(relation_weight, rel_g)
    res_lo, res_hi = _edge_math(head_rows, tail_rows, rel_rows)
    lo_p = _sc_scatter_add(res_lo, head_s, ENT_PAD, z_ent32)
    hi_p = _sc_scatter_add(res_hi, head_s, ENT_PAD, z_ent32)
    cnt_p = _sc_count(head_s, E_PAD, ENT_PAD, z_ent16, ones16)
    entity_agg = _finalize_entity(lo_p[:, :N_ENTITIES], hi_p[:, :N_ENTITIES],
                                  cnt_p[:, :N_ENTITIES])

    user_rows = _sc_gather(user_emb, mrow_g)
    itm_p = _sc_scatter_add(user_rows, mcol_s, ITM_PAD, z_itm)
    item_agg_cf = _sum_partials(itm_p[:, :N_ITEMS], N_ITEMS)

    fusion = _gate_fusion(item_emb_cf, entity_emb[:N_ITEMS], gate1_w, gate2_w)
    fus_rows = _sc_gather(fusion, mcol_g)
    scaled = _scale_rows(fus_rows, val_p)
    usr_p = _sc_scatter_add(scaled, mrow_s, USR_PAD, z_usr)
    user_agg = _sum_partials(usr_p[:, :N_USERS], N_USERS)

    return (entity_agg, user_agg, item_agg_cf)


# speedup vs baseline: 2.3798x; 1.0346x over previous
import functools

import jax
import jax.numpy as jnp
from jax import lax
from jax.experimental import pallas as pl
from jax.experimental.pallas import tpu as pltpu
from jax.experimental.pallas import tpu_sc as plsc

N_ENTITIES = 50000
N_USERS = 30000
N_ITEMS = 20000
N_EDGES = 800000
NNZ = 600000
DIM = 64
MIN_NORM = 1e-15
EPS = 1e-5

NC = 2
NS = 16
NW = NC * NS
CHUNK = 128
KMAX = 7


def _pad_to(n, m):
    return ((n + m - 1) // m) * m


E_PAD = _pad_to(N_EDGES, NW * CHUNK * KMAX)
Z_PAD = _pad_to(NNZ, NW * CHUNK * KMAX)
ENT_PAD = _pad_to(N_ENTITIES + 1, NS)
ITM_PAD = _pad_to(N_ITEMS + 1, NS)
USR_PAD = _pad_to(N_USERS + 1, NS)

_SC_PARAMS = dict(
    compiler_params=pltpu.CompilerParams(use_tc_tiling_on_sc=False),
)


def _mesh():
    return plsc.VectorSubcoreMesh(core_axis_name="c", subcore_axis_name="s")


def _sc_gather(table, idx2, K=7):
    n = idx2.shape[0] * CHUNK
    d = table.shape[1]
    per_w = n // NW
    ngrp = per_w // (CHUNK * K)

    @functools.partial(
        pl.kernel, mesh=_mesh(),
        out_type=jax.ShapeDtypeStruct((n, d), jnp.float32),
        scratch_types=[
            pltpu.VMEM((K, CHUNK), jnp.int32),
            pltpu.VMEM((K * CHUNK, d), jnp.float32),
            pltpu.SemaphoreType.DMA,
        ],
        **_SC_PARAMS,
    )
    def k(table_h, idx_h, out_h, idx_v, rows_v, sem):
        wid = lax.axis_index("s") * NC + lax.axis_index("c")
        base = wid * per_w

        def step(g, carry):
            off = base + g * (CHUNK * K)
            pltpu.sync_copy(idx_h.at[pl.ds(off // CHUNK, K)], idx_v)
            cps = [pltpu.async_copy(table_h.at[idx_v.at[b]],
                                    rows_v.at[pl.ds(b * CHUNK, CHUNK)], sem)
                   for b in range(K)]
            for cp in cps:
                cp.wait()
            pltpu.sync_copy(rows_v, out_h.at[pl.ds(off, CHUNK * K)])
            return carry

        lax.fori_loop(0, ngrp, step, 0)

    return k(table, idx2)


def _sc_scatter_add(src, idx2, nseg_pad, zeros_h, K=7):
    n, d = src.shape
    per_w = n // NW
    ngrp = per_w // (CHUNK * K)
    rz = nseg_pad // NS

    @functools.partial(
        pl.kernel, mesh=_mesh(),
        out_type=jax.ShapeDtypeStruct((NC, nseg_pad, d), jnp.float32),
        scratch_types=[
            pltpu.VMEM((K, CHUNK), jnp.int32),
            pltpu.VMEM((K * CHUNK, d), jnp.float32),
            pltpu.VMEM_SHARED((nseg_pad, d), jnp.float32),
        ],
        **_SC_PARAMS,
    )
    def k(src_h, idx_h, z_h, out_h, idx_v, rows_v, acc):
        cid = lax.axis_index("c")
        sid = lax.axis_index("s")
        wid = sid * NC + cid
        base = wid * per_w
        pltpu.sync_copy(z_h.at[pl.ds(sid * rz, rz)], acc.at[pl.ds(sid * rz, rz)])
        plsc.subcore_barrier()

        def step(g, carry):
            off = base + g * (CHUNK * K)
            pltpu.sync_copy(idx_h.at[pl.ds(off // CHUNK, K)], idx_v)
            pltpu.sync_copy(src_h.at[pl.ds(off, CHUNK * K)], rows_v)
            for b in range(K):
                pltpu.sync_copy(rows_v.at[pl.ds(b * CHUNK, CHUNK)],
                                acc.at[idx_v.at[b]], add=True)
            return carry

        lax.fori_loop(0, ngrp, step, 0)
        plsc.subcore_barrier()
        pltpu.sync_copy(acc.at[pl.ds(sid * rz, rz)],
                        out_h.at[cid, pl.ds(sid * rz, rz)])

    return k(src, idx2, zeros_h)


def _sc_gather_scatter_add(table, gidx2, sidx2, nseg_pad, zeros_h, K=3):
    n = gidx2.shape[0] * CHUNK
    d = table.shape[1]
    per_w = n // NW
    ngrp = per_w // (CHUNK * K)
    rz = nseg_pad // NS

    @functools.partial(
        pl.kernel, mesh=_mesh(),
        out_type=jax.ShapeDtypeStruct((NC, nseg_pad, d), jnp.float32),
        scratch_types=[
            pltpu.VMEM((K, CHUNK), jnp.int32),
            pltpu.VMEM((K, CHUNK), jnp.int32),
            pltpu.VMEM((K * CHUNK, d), jnp.float32),
            pltpu.VMEM_SHARED((nseg_pad, d), jnp.float32),
            pltpu.SemaphoreType.DMA,
        ],
        **_SC_PARAMS,
    )
    def k(table_h, gidx_h, sidx_h, z_h, out_h, gidx_v, sidx_v, rows_v, acc, sem):
        cid = lax.axis_index("c")
        sid = lax.axis_index("s")
        wid = sid * NC + cid
        base = wid * per_w
        pltpu.sync_copy(z_h.at[pl.ds(sid * rz, rz)], acc.at[pl.ds(sid * rz, rz)])
        plsc.subcore_barrier()

        def step(g, carry):
            off = base + g * (CHUNK * K)
            pltpu.sync_copy(gidx_h.at[pl.ds(off // CHUNK, K)], gidx_v)
            pltpu.sync_copy(sidx_h.at[pl.ds(off // CHUNK, K)], sidx_v)
            cps = [pltpu.async_copy(table_h.at[gidx_v.at[b]],
                                    rows_v.at[pl.ds(b * CHUNK, CHUNK)], sem)
                   for b in range(K)]
            for cp in cps:
                cp.wait()
            for b in range(K):
                pltpu.sync_copy(rows_v.at[pl.ds(b * CHUNK, CHUNK)],
                                acc.at[sidx_v.at[b]], add=True)
            return carry

        lax.fori_loop(0, ngrp, step, 0)
        plsc.subcore_barrier()
        pltpu.sync_copy(acc.at[pl.ds(sid * rz, rz)],
                        out_h.at[cid, pl.ds(sid * rz, rz)])

    return k(table, gidx2, sidx2, zeros_h)


def _sc_count(idx2, n, nseg_pad, zeros_h, ones_h, K=7):
    per_w = n // NW
    ngrp = per_w // (CHUNK * K)
    rz = nseg_pad // NS

    @functools.partial(
        pl.kernel, mesh=_mesh(),
        out_type=jax.ShapeDtypeStruct((NC, nseg_pad, 16), jnp.float32),
        scratch_types=[
            pltpu.VMEM((K, CHUNK), jnp.int32),
            pltpu.VMEM((CHUNK, 16), jnp.float32),
            pltpu.VMEM_SHARED((nseg_pad, 16), jnp.float32),
        ],
        **_SC_PARAMS,
    )
    def k(idx_h, z_h, ones_hh, out_h, idx_v, ones_v, acc):
        cid = lax.axis_index("c")
        sid = lax.axis_index("s")
        wid = sid * NC + cid
        base = wid * per_w
        pltpu.sync_copy(ones_hh, ones_v)
        pltpu.sync_copy(z_h.at[pl.ds(sid * rz, rz)], acc.at[pl.ds(sid * rz, rz)])
        plsc.subcore_barrier()

        def step(g, carry):
            off = base + g * (CHUNK * K)
            pltpu.sync_copy(idx_h.at[pl.ds(off // CHUNK, K)], idx_v)
            for b in range(K):
                pltpu.sync_copy(ones_v, acc.at[idx_v.at[b]], add=True)
            return carry

        lax.fori_loop(0, ngrp, step, 0)
        plsc.subcore_barrier()
        pltpu.sync_copy(acc.at[pl.ds(sid * rz, rz)],
                        out_h.at[cid, pl.ds(sid * rz, rz)])

    return k(idx2, zeros_h, ones_h)


def _norm(x):
    return jnp.clip(jnp.sqrt(jnp.sum(x * x, axis=-1, keepdims=True)), MIN_NORM, None)


def _artanh(x):
    x = jnp.clip(x, -1.0 + 1e-7, 1.0 - 1e-7)
    return 0.5 * jnp.log((1.0 + x) / (1.0 - x))


def _project(x):
    norm = _norm(x)
    maxnorm = 1.0 - EPS
    return jnp.where(norm > maxnorm, x / norm * maxnorm, x)


def _expmap0(u):
    norm = _norm(u)
    return _project(jnp.tanh(norm) * u / norm)


def _mobius_add(x, y):
    x2 = jnp.sum(x * x, axis=-1, keepdims=True)
    y2 = jnp.sum(y * y, axis=-1, keepdims=True)
    xy = jnp.sum(x * y, axis=-1, keepdims=True)
    num = (1.0 + 2.0 * xy + y2) * x + (1.0 - x2) * y
    den = 1.0 + 2.0 * xy + x2 * y2
    return num / jnp.clip(den, MIN_NORM, None)


def _lambda_x(x):
    x2 = jnp.sum(x * x, axis=-1, keepdims=True)
    return 2.0 / jnp.clip(1.0 - x2, MIN_NORM, None)


def _expmap(u, p):
    nu = _norm(u)
    second = jnp.tanh(_lambda_x(p) * nu / 2.0) * u / nu
    return _project(_mobius_add(p, second))


def _logmap(y, x):
    sub = _mobius_add(-x, y)
    ns = _norm(sub)
    return 2.0 / _lambda_x(x) * _artanh(ns) * sub / ns


def _edge_math(head_rows, tail_rows, rel_rows):
    blk = 1024
    grid = E_PAD // blk

    def body(h_ref, t_ref, r_ref, lo_ref, hi_ref):
        h = h_ref[...]
        t = t_ref[...]
        r = r_ref[...]
        hh = _expmap0(h)
        ht = _expmap(t, hh)
        hr = _expmap(r, hh)
        res = _project(_mobius_add(ht, hr))
        res = _logmap(res, hh)
        lo_ref[...] = res[:, :32]
        hi_ref[...] = res[:, 32:]

    spec_in = pl.BlockSpec((blk, DIM), lambda i: (i, 0))
    spec_out = pl.BlockSpec((blk, 32), lambda i: (i, 0))
    return pl.pallas_call(
        body, grid=(grid,),
        in_specs=[spec_in, spec_in, spec_in],
        out_specs=[spec_out, spec_out],
        out_shape=[jax.ShapeDtypeStruct((E_PAD, 32), jnp.float32)] * 2,
    )(head_rows, tail_rows, rel_rows)


def _gate_fusion(item_cf, item_kg, g1, g2):
    blk = 1000
    grid = N_ITEMS // blk

    def body(cf_ref, kg_ref, g1_ref, g2_ref, out_ref):
        cf = cf_ref[...]
        kg = kg_ref[...]
        y = lax.dot_general(cf, g1_ref[...], (((1,), (1,)), ((), ())),
                            preferred_element_type=jnp.float32)
        y += lax.dot_general(kg, g2_ref[...], (((1,), (1,)), ((), ())),
                             preferred_element_type=jnp.float32)
        gi = jax.nn.sigmoid(y)
        out_ref[...] = gi * cf + (1.0 - gi) * kg

    spec = pl.BlockSpec((blk, DIM), lambda i: (i, 0))
    wspec = pl.BlockSpec((DIM, DIM), lambda i: (0, 0))
    return pl.pallas_call(
        body, grid=(grid,),
        in_specs=[spec, spec, wspec, wspec],
        out_specs=spec,
        out_shape=jax.ShapeDtypeStruct((N_ITEMS, DIM), jnp.float32),
    )(item_cf, item_kg, g1, g2)


def _scale_rows(rows, vals):
    blk = 2048
    grid = Z_PAD // blk

    def body(r_ref, v_ref, o_ref):
        o_ref[...] = r_ref[...] * v_ref[...]

    return pl.pallas_call(
        body, grid=(grid,),
        in_specs=[pl.BlockSpec((blk, DIM), lambda i: (i, 0)),
                  pl.BlockSpec((blk, 1), lambda i: (i, 0))],
        out_specs=pl.BlockSpec((blk, DIM), lambda i: (i, 0)),
        out_shape=jax.ShapeDtypeStruct((Z_PAD, DIM), jnp.float32),
    )(rows, vals)


def _finalize_entity(lo_p, hi_p, cnt_p):
    blk = 1000
    grid = N_ENTITIES // blk

    def body(lo_ref, hi_ref, c_ref, o_ref):
        lo = lo_ref[0] + lo_ref[1]
        hi = hi_ref[0] + hi_ref[1]
        c = c_ref[0, :, 0:1] + c_ref[1, :, 0:1]
        s = jnp.concatenate([lo, hi], axis=1)
        o_ref[...] = s / jnp.clip(c, 1.0, None)

    return pl.pallas_call(
        body, grid=(grid,),
        in_specs=[pl.BlockSpec((NC, blk, 32), lambda i: (0, i, 0)),
                  pl.BlockSpec((NC, blk, 32), lambda i: (0, i, 0)),
                  pl.BlockSpec((NC, blk, 16), lambda i: (0, i, 0))],
        out_specs=pl.BlockSpec((blk, DIM), lambda i: (i, 0)),
        out_shape=jax.ShapeDtypeStruct((N_ENTITIES, DIM), jnp.float32),
    )(lo_p, hi_p, cnt_p)


def _sum_partials(p, nrows):
    blk = 1000
    grid = nrows // blk

    def body(p_ref, o_ref):
        o_ref[...] = p_ref[0] + p_ref[1]

    return pl.pallas_call(
        body, grid=(grid,),
        in_specs=[pl.BlockSpec((NC, blk, DIM), lambda i: (0, i, 0))],
        out_specs=pl.BlockSpec((blk, DIM), lambda i: (i, 0)),
        out_shape=jax.ShapeDtypeStruct((nrows, DIM), jnp.float32),
    )(p)


def _pad2(idx, n_pad, fill):
    p = n_pad - idx.shape[0]
    return jnp.concatenate(
        [idx, jnp.full((p,), fill, jnp.int32)]).reshape(n_pad // CHUNK, CHUNK)


def kernel(entity_emb, user_emb, item_emb_cf, edge_index, edge_type,
           mat_row, mat_col, mat_val, relation_weight, gate1_w, gate2_w):
    i32 = jnp.int32
    head = edge_index[0].astype(i32)
    tail = edge_index[1].astype(i32)
    rel = (edge_type - 1).astype(i32)
    mrow = mat_row.astype(i32)
    mcol = mat_col.astype(i32)

    head_g = _pad2(head, E_PAD, 0)
    tail_g = _pad2(tail, E_PAD, 0)
    rel_g = _pad2(rel, E_PAD, 0)
    head_s = _pad2(head, E_PAD, N_ENTITIES)
    mrow_g = _pad2(mrow, Z_PAD, 0)
    mcol_g = _pad2(mcol, Z_PAD, 0)
    mrow_s = _pad2(mrow, Z_PAD, N_USERS)
    mcol_s = _pad2(mcol, Z_PAD, N_ITEMS)
    zp = Z_PAD - NNZ
    val_p = jnp.concatenate([mat_val, jnp.zeros((zp,), jnp.float32)])[:, None]

    z_ent32 = jnp.zeros((ENT_PAD, 32), jnp.float32)
    z_ent16 = jnp.zeros((ENT_PAD, 16), jnp.float32)
    z_itm = jnp.zeros((ITM_PAD, DIM), jnp.float32)
    z_usr = jnp.zeros((USR_PAD, DIM), jnp.float32)
    ones16 = jnp.ones((CHUNK, 16), jnp.float32)

    head_rows = _sc_gather(entity_emb, head_g)
    tail_rows = _sc_gather(entity_emb, tail_g)
    rel_rows = _sc_gather(relation_weight, rel_g)
    res_lo, res_hi = _edge_math(head_rows, tail_rows, rel_rows)
    lo_p = _sc_scatter_add(res_lo, head_s, ENT_PAD, z_ent32)
    hi_p = _sc_scatter_add(res_hi, head_s, ENT_PAD, z_ent32)
    cnt_p = _sc_count(head_s, E_PAD, ENT_PAD, z_ent16, ones16)
    entity_agg = _finalize_entity(lo_p[:, :N_ENTITIES], hi_p[:, :N_ENTITIES],
                                  cnt_p[:, :N_ENTITIES])

    itm_p = _sc_gather_scatter_add(user_emb, mrow_g, mcol_s, ITM_PAD, z_itm)
    item_agg_cf = _sum_partials(itm_p[:, :N_ITEMS], N_ITEMS)

    fusion = _gate_fusion(item_emb_cf, entity_emb[:N_ITEMS], gate1_w, gate2_w)
    fus_rows = _sc_gather(fusion, mcol_g)
    scaled = _scale_rows(fus_rows, val_p)
    usr_p = _sc_scatter_add(scaled, mrow_s, USR_PAD, z_usr, K=1)
    user_agg = _sum_partials(usr_p[:, :N_USERS], N_USERS)

    return (entity_agg, user_agg, item_agg_cf)


# speedup vs baseline: 2.3817x; 1.0008x over previous
import functools

import jax
import jax.numpy as jnp
from jax import lax
from jax.experimental import pallas as pl
from jax.experimental.pallas import tpu as pltpu
from jax.experimental.pallas import tpu_sc as plsc

N_ENTITIES = 50000
N_USERS = 30000
N_ITEMS = 20000
N_EDGES = 800000
NNZ = 600000
DIM = 64
MIN_NORM = 1e-15
EPS = 1e-5

NC = 2
NS = 16
NW = NC * NS
CHUNK = 128
KMAX = 7


def _pad_to(n, m):
    return ((n + m - 1) // m) * m


E_PAD = _pad_to(N_EDGES, NW * CHUNK * KMAX)
Z_PAD = _pad_to(NNZ, NW * CHUNK * KMAX)
ENT_PAD = _pad_to(N_ENTITIES + 1, NS)
ITM_PAD = _pad_to(N_ITEMS + 1, NS)
USR_PAD = _pad_to(N_USERS + 1, NS)

_SC_PARAMS = dict(
    compiler_params=pltpu.CompilerParams(use_tc_tiling_on_sc=False),
)


def _mesh():
    return plsc.VectorSubcoreMesh(core_axis_name="c", subcore_axis_name="s")


def _sc_gather(table, idx2, K=7):
    n = idx2.shape[0] * CHUNK
    d = table.shape[1]
    per_w = n // NW
    ngrp = per_w // (CHUNK * K)

    @functools.partial(
        pl.kernel, mesh=_mesh(),
        out_type=jax.ShapeDtypeStruct((n, d), jnp.float32),
        scratch_types=[
            pltpu.VMEM((K, CHUNK), jnp.int32),
            pltpu.VMEM((K * CHUNK, d), jnp.float32),
            pltpu.SemaphoreType.DMA,
        ],
        **_SC_PARAMS,
    )
    def k(table_h, idx_h, out_h, idx_v, rows_v, sem):
        wid = lax.axis_index("s") * NC + lax.axis_index("c")
        base = wid * per_w

        def step(g, carry):
            off = base + g * (CHUNK * K)
            pltpu.sync_copy(idx_h.at[pl.ds(off // CHUNK, K)], idx_v)
            cps = [pltpu.async_copy(table_h.at[idx_v.at[b]],
                                    rows_v.at[pl.ds(b * CHUNK, CHUNK)], sem)
                   for b in range(K)]
            for cp in cps:
                cp.wait()
            pltpu.sync_copy(rows_v, out_h.at[pl.ds(off, CHUNK * K)])
            return carry

        lax.fori_loop(0, ngrp, step, 0)

    return k(table, idx2)


def _sc_scatter_add(src, idx2, nseg_pad, zeros_h, K=7):
    n, d = src.shape
    per_w = n // NW
    ngrp = per_w // (CHUNK * K)
    rz = nseg_pad // NS

    @functools.partial(
        pl.kernel, mesh=_mesh(),
        out_type=jax.ShapeDtypeStruct((NC, nseg_pad, d), jnp.float32),
        scratch_types=[
            pltpu.VMEM((K, CHUNK), jnp.int32),
            pltpu.VMEM((K * CHUNK, d), jnp.float32),
            pltpu.VMEM_SHARED((nseg_pad, d), jnp.float32),
            pltpu.SemaphoreType.DMA,
        ],
        **_SC_PARAMS,
    )
    def k(src_h, idx_h, z_h, out_h, idx_v, rows_v, acc, sem):
        cid = lax.axis_index("c")
        sid = lax.axis_index("s")
        wid = sid * NC + cid
        base = wid * per_w
        pltpu.sync_copy(z_h.at[pl.ds(sid * rz, rz)], acc.at[pl.ds(sid * rz, rz)])
        plsc.subcore_barrier()

        def step(g, carry):
            off = base + g * (CHUNK * K)
            pltpu.sync_copy(idx_h.at[pl.ds(off // CHUNK, K)], idx_v)
            pltpu.sync_copy(src_h.at[pl.ds(off, CHUNK * K)], rows_v)
            cps = [pltpu.async_copy(rows_v.at[pl.ds(b * CHUNK, CHUNK)],
                                    acc.at[idx_v.at[b]], sem, add=True)
                   for b in range(K)]
            for cp in cps:
                cp.wait()
            return carry

        lax.fori_loop(0, ngrp, step, 0)
        plsc.subcore_barrier()
        pltpu.sync_copy(acc.at[pl.ds(sid * rz, rz)],
                        out_h.at[cid, pl.ds(sid * rz, rz)])

    return k(src, idx2, zeros_h)


def _sc_gather_scatter_add(table, gidx2, sidx2, nseg_pad, zeros_h, K=3):
    n = gidx2.shape[0] * CHUNK
    d = table.shape[1]
    per_w = n // NW
    ngrp = per_w // (CHUNK * K)
    rz = nseg_pad // NS

    @functools.partial(
        pl.kernel, mesh=_mesh(),
        out_type=jax.ShapeDtypeStruct((NC, nseg_pad, d), jnp.float32),
        scratch_types=[
            pltpu.VMEM((K, CHUNK), jnp.int32),
            pltpu.VMEM((K, CHUNK), jnp.int32),
            pltpu.VMEM((K * CHUNK, d), jnp.float32),
            pltpu.VMEM_SHARED((nseg_pad, d), jnp.float32),
            pltpu.SemaphoreType.DMA,
        ],
        **_SC_PARAMS,
    )
    def k(table_h, gidx_h, sidx_h, z_h, out_h, gidx_v, sidx_v, rows_v, acc, sem):
        cid = lax.axis_index("c")
        sid = lax.axis_index("s")
        wid = sid * NC + cid
        base = wid * per_w
        pltpu.sync_copy(z_h.at[pl.ds(sid * rz, rz)], acc.at[pl.ds(sid * rz, rz)])
        plsc.subcore_barrier()

        def step(g, carry):
            off = base + g * (CHUNK * K)
            pltpu.sync_copy(gidx_h.at[pl.ds(off // CHUNK, K)], gidx_v)
            pltpu.sync_copy(sidx_h.at[pl.ds(off // CHUNK, K)], sidx_v)
            cps = [pltpu.async_copy(table_h.at[gidx_v.at[b]],
                                    rows_v.at[pl.ds(b * CHUNK, CHUNK)], sem)
                   for b in range(K)]
            for cp in cps:
                cp.wait()
            aps = [pltpu.async_copy(rows_v.at[pl.ds(b * CHUNK, CHUNK)],
                                    acc.at[sidx_v.at[b]], sem, add=True)
                   for b in range(K)]
            for ap in aps:
                ap.wait()
            return carry

        lax.fori_loop(0, ngrp, step, 0)
        plsc.subcore_barrier()
        pltpu.sync_copy(acc.at[pl.ds(sid * rz, rz)],
                        out_h.at[cid, pl.ds(sid * rz, rz)])

    return k(table, gidx2, sidx2, zeros_h)


def _sc_count(idx2, n, nseg_pad, zeros_h, ones_h, K=7):
    per_w = n // NW
    ngrp = per_w // (CHUNK * K)
    rz = nseg_pad // NS

    @functools.partial(
        pl.kernel, mesh=_mesh(),
        out_type=jax.ShapeDtypeStruct((NC, nseg_pad, 16), jnp.float32),
        scratch_types=[
            pltpu.VMEM((K, CHUNK), jnp.int32),
            pltpu.VMEM((CHUNK, 16), jnp.float32),
            pltpu.VMEM_SHARED((nseg_pad, 16), jnp.float32),
        ],
        **_SC_PARAMS,
    )
    def k(idx_h, z_h, ones_hh, out_h, idx_v, ones_v, acc):
        cid = lax.axis_index("c")
        sid = lax.axis_index("s")
        wid = sid * NC + cid
        base = wid * per_w
        pltpu.sync_copy(ones_hh, ones_v)
        pltpu.sync_copy(z_h.at[pl.ds(sid * rz, rz)], acc.at[pl.ds(sid * rz, rz)])
        plsc.subcore_barrier()

        def step(g, carry):
            off = base + g * (CHUNK * K)
            pltpu.sync_copy(idx_h.at[pl.ds(off // CHUNK, K)], idx_v)
            for b in range(K):
                pltpu.sync_copy(ones_v, acc.at[idx_v.at[b]], add=True)
            return carry

        lax.fori_loop(0, ngrp, step, 0)
        plsc.subcore_barrier()
        pltpu.sync_copy(acc.at[pl.ds(sid * rz, rz)],
                        out_h.at[cid, pl.ds(sid * rz, rz)])

    return k(idx2, zeros_h, ones_h)


def _norm(x):
    return jnp.clip(jnp.sqrt(jnp.sum(x * x, axis=-1, keepdims=True)), MIN_NORM, None)


def _artanh(x):
    x = jnp.clip(x, -1.0 + 1e-7, 1.0 - 1e-7)
    return 0.5 * jnp.log((1.0 + x) / (1.0 - x))


def _project(x):
    norm = _norm(x)
    maxnorm = 1.0 - EPS
    return jnp.where(norm > maxnorm, x / norm * maxnorm, x)


def _expmap0(u):
    norm = _norm(u)
    return _project(jnp.tanh(norm) * u / norm)


def _mobius_add(x, y):
    x2 = jnp.sum(x * x, axis=-1, keepdims=True)
    y2 = jnp.sum(y * y, axis=-1, keepdims=True)
    xy = jnp.sum(x * y, axis=-1, keepdims=True)
    num = (1.0 + 2.0 * xy + y2) * x + (1.0 - x2) * y
    den = 1.0 + 2.0 * xy + x2 * y2
    return num / jnp.clip(den, MIN_NORM, None)


def _lambda_x(x):
    x2 = jnp.sum(x * x, axis=-1, keepdims=True)
    return 2.0 / jnp.clip(1.0 - x2, MIN_NORM, None)


def _expmap(u, p):
    nu = _norm(u)
    second = jnp.tanh(_lambda_x(p) * nu / 2.0) * u / nu
    return _project(_mobius_add(p, second))


def _logmap(y, x):
    sub = _mobius_add(-x, y)
    ns = _norm(sub)
    return 2.0 / _lambda_x(x) * _artanh(ns) * sub / ns


def _edge_math(head_rows, tail_rows, rel_rows):
    blk = 1024
    grid = E_PAD // blk

    def body(h_ref, t_ref, r_ref, lo_ref, hi_ref):
        h = h_ref[...]
        t = t_ref[...]
        r = r_ref[...]
        hh = _expmap0(h)
        ht = _expmap(t, hh)
        hr = _expmap(r, hh)
        res = _project(_mobius_add(ht, hr))
        res = _logmap(res, hh)
        lo_ref[...] = res[:, :32]
        hi_ref[...] = res[:, 32:]

    spec_in = pl.BlockSpec((blk, DIM), lambda i: (i, 0))
    spec_out = pl.BlockSpec((blk, 32), lambda i: (i, 0))
    return pl.pallas_call(
        body, grid=(grid,),
        in_specs=[spec_in, spec_in, spec_in],
        out_specs=[spec_out, spec_out],
        out_shape=[jax.ShapeDtypeStruct((E_PAD, 32), jnp.float32)] * 2,
    )(head_rows, tail_rows, rel_rows)


def _gate_fusion(item_cf, item_kg, g1, g2):
    blk = 1000
    grid = N_ITEMS // blk

    def body(cf_ref, kg_ref, g1_ref, g2_ref, out_ref):
        cf = cf_ref[...]
        kg = kg_ref[...]
        y = lax.dot_general(cf, g1_ref[...], (((1,), (1,)), ((), ())),
                            preferred_element_type=jnp.float32)
        y += lax.dot_general(kg, g2_ref[...], (((1,), (1,)), ((), ())),
                             preferred_element_type=jnp.float32)
        gi = jax.nn.sigmoid(y)
        out_ref[...] = gi * cf + (1.0 - gi) * kg

    spec = pl.BlockSpec((blk, DIM), lambda i: (i, 0))
    wspec = pl.BlockSpec((DIM, DIM), lambda i: (0, 0))
    return pl.pallas_call(
        body, grid=(grid,),
        in_specs=[spec, spec, wspec, wspec],
        out_specs=spec,
        out_shape=jax.ShapeDtypeStruct((N_ITEMS, DIM), jnp.float32),
    )(item_cf, item_kg, g1, g2)


def _scale_rows(rows, vals):
    blk = 2048
    grid = Z_PAD // blk

    def body(r_ref, v_ref, o_ref):
        o_ref[...] = r_ref[...] * v_ref[...]

    return pl.pallas_call(
        body, grid=(grid,),
        in_specs=[pl.BlockSpec((blk, DIM), lambda i: (i, 0)),
                  pl.BlockSpec((blk, 1), lambda i: (i, 0))],
        out_specs=pl.BlockSpec((blk, DIM), lambda i: (i, 0)),
        out_shape=jax.ShapeDtypeStruct((Z_PAD, DIM), jnp.float32),
    )(rows, vals)


def _finalize_entity(lo_p, hi_p, cnt_p):
    blk = 1000
    grid = N_ENTITIES // blk

    def body(lo_ref, hi_ref, c_ref, o_ref):
        lo = lo_ref[0] + lo_ref[1]
        hi = hi_ref[0] + hi_ref[1]
        c = c_ref[0, :, 0:1] + c_ref[1, :, 0:1]
        s = jnp.concatenate([lo, hi], axis=1)
        o_ref[...] = s / jnp.clip(c, 1.0, None)

    return pl.pallas_call(
        body, grid=(grid,),
        in_specs=[pl.BlockSpec((NC, blk, 32), lambda i: (0, i, 0)),
                  pl.BlockSpec((NC, blk, 32), lambda i: (0, i, 0)),
                  pl.BlockSpec((NC, blk, 16), lambda i: (0, i, 0))],
        out_specs=pl.BlockSpec((blk, DIM), lambda i: (i, 0)),
        out_shape=jax.ShapeDtypeStruct((N_ENTITIES, DIM), jnp.float32),
    )(lo_p, hi_p, cnt_p)


def _sum_partials(p, nrows):
    blk = 1000
    grid = nrows // blk

    def body(p_ref, o_ref):
        o_ref[...] = p_ref[0] + p_ref[1]

    return pl.pallas_call(
        body, grid=(grid,),
        in_specs=[pl.BlockSpec((NC, blk, DIM), lambda i: (0, i, 0))],
        out_specs=pl.BlockSpec((blk, DIM), lambda i: (i, 0)),
        out_shape=jax.ShapeDtypeStruct((nrows, DIM), jnp.float32),
    )(p)


def _pad2(idx, n_pad, fill):
    p = n_pad - idx.shape[0]
    return jnp.concatenate(
        [idx, jnp.full((p,), fill, jnp.int32)]).reshape(n_pad // CHUNK, CHUNK)


def kernel(entity_emb, user_emb, item_emb_cf, edge_index, edge_type,
           mat_row, mat_col, mat_val, relation_weight, gate1_w, gate2_w):
    i32 = jnp.int32
    head = edge_index[0].astype(i32)
    tail = edge_index[1].astype(i32)
    rel = (edge_type - 1).astype(i32)
    mrow = mat_row.astype(i32)
    mcol = mat_col.astype(i32)

    head_g = _pad2(head, E_PAD, 0)
    tail_g = _pad2(tail, E_PAD, 0)
    rel_g = _pad2(rel, E_PAD, 0)
    head_s = _pad2(head, E_PAD, N_ENTITIES)
    mrow_g = _pad2(mrow, Z_PAD, 0)
    mcol_g = _pad2(mcol, Z_PAD, 0)
    mrow_s = _pad2(mrow, Z_PAD, N_USERS)
    mcol_s = _pad2(mcol, Z_PAD, N_ITEMS)
    zp = Z_PAD - NNZ
    val_p = jnp.concatenate([mat_val, jnp.zeros((zp,), jnp.float32)])[:, None]

    z_ent32 = jnp.zeros((ENT_PAD, 32), jnp.float32)
    z_ent16 = jnp.zeros((ENT_PAD, 16), jnp.float32)
    z_itm = jnp.zeros((ITM_PAD, DIM), jnp.float32)
    z_usr = jnp.zeros((USR_PAD, DIM), jnp.float32)
    ones16 = jnp.ones((CHUNK, 16), jnp.float32)

    head_rows = _sc_gather(entity_emb, head_g)
    tail_rows = _sc_gather(entity_emb, tail_g)
    rel_rows = _sc_gather(relation_weight, rel_g)
    res_lo, res_hi = _edge_math(head_rows, tail_rows, rel_rows)
    lo_p = _sc_scatter_add(res_lo, head_s, ENT_PAD, z_ent32)
    hi_p = _sc_scatter_add(res_hi, head_s, ENT_PAD, z_ent32)
    cnt_p = _sc_count(head_s, E_PAD, ENT_PAD, z_ent16, ones16)
    entity_agg = _finalize_entity(lo_p[:, :N_ENTITIES], hi_p[:, :N_ENTITIES],
                                  cnt_p[:, :N_ENTITIES])

    itm_p = _sc_gather_scatter_add(user_emb, mrow_g, mcol_s, ITM_PAD, z_itm)
    item_agg_cf = _sum_partials(itm_p[:, :N_ITEMS], N_ITEMS)

    fusion = _gate_fusion(item_emb_cf, entity_emb[:N_ITEMS], gate1_w, gate2_w)
    fus_rows = _sc_gather(fusion, mcol_g)
    scaled = _scale_rows(fus_rows, val_p)
    usr_p = _sc_scatter_add(scaled, mrow_s, USR_PAD, z_usr, K=1)
    user_agg = _sum_partials(usr_p[:, :N_USERS], N_USERS)

    return (entity_agg, user_agg, item_agg_cf)
